# baseline (speedup 1.0000x reference)
import jax
import jax.numpy as jnp
from jax.experimental import pallas as pl
from jax.experimental.pallas import tpu as pltpu

N_BLOCK = 1024


def _assign_body(state_ref, cent_ref, idx_ref, dist_ref):
    a = state_ref[:]
    c = cent_ref[:]
    x2 = jnp.sum(a * a, axis=1, keepdims=True)
    c2 = jnp.sum(c * c, axis=1)[None, :]
    mm = jax.lax.dot_general(
        a, c, (((1,), (1,)), ((), ())),
        preferred_element_type=jnp.float32)
    d2 = (x2 + c2) - 2.0 * mm
    m2 = jnp.min(d2, axis=1)
    m = jnp.sqrt(jnp.clip(m2, 0.0, None))
    dist_ref[:] = m
    mbits = jax.lax.bitcast_convert_type(m * m, jnp.int32)
    t = jnp.full_like(m, -jnp.inf)
    for e in range(-4, 5):
        xe = jax.lax.bitcast_convert_type(mbits + e, jnp.float32)
        ok = jnp.sqrt(xe) <= m
        t = jnp.where(ok, jnp.maximum(t, xe), t)
    t = jnp.maximum(t, m2)
    iota = jax.lax.broadcasted_iota(jnp.int32, d2.shape, 1).astype(jnp.float32)
    hit = jnp.where(d2 <= t[:, None], iota, float(d2.shape[1]))
    idx_ref[:] = jnp.min(hit, axis=1).astype(jnp.int32)


def kernel(state, centroids):
    n, d = state.shape
    k, _ = centroids.shape
    grid = (n // N_BLOCK,)
    idx, dist = pl.pallas_call(
        _assign_body,
        grid=grid,
        in_specs=[
            pl.BlockSpec((N_BLOCK, d), lambda i: (i, 0)),
            pl.BlockSpec((k, d), lambda i: (0, 0)),
        ],
        out_specs=[
            pl.BlockSpec((N_BLOCK,), lambda i: (i,)),
            pl.BlockSpec((N_BLOCK,), lambda i: (i,)),
        ],
        out_shape=[
            jax.ShapeDtypeStruct((n,), jnp.int32),
            jax.ShapeDtypeStruct((n,), jnp.float32),
        ],
        compiler_params=pltpu.CompilerParams(
            dimension_semantics=("arbitrary",)),
    )(state, centroids)
    return (idx, dist)


# speedup vs baseline: 1.1605x; 1.1605x over previous
import jax
import jax.numpy as jnp
from jax.experimental import pallas as pl
from jax.experimental.pallas import tpu as pltpu

N_BLOCK = 1024


def _assign_body(state_ref, cent_ref, idx_ref, dist_ref):
    a = state_ref[:]
    c = cent_ref[:]
    x2 = jnp.sum(a * a, axis=1, keepdims=True)
    c2 = jnp.sum(c * c, axis=1)[None, :]
    mm = jax.lax.dot_general(
        a, c, (((1,), (1,)), ((), ())),
        preferred_element_type=jnp.float32)
    d = jnp.sqrt(jnp.clip((x2 + c2) - 2.0 * mm, 0.0, None))
    m = jnp.min(d, axis=1)
    dist_ref[:] = m
    iota = jax.lax.broadcasted_iota(jnp.int32, d.shape, 1).astype(jnp.float32)
    hit = jnp.where(d == m[:, None], iota, float(d.shape[1]))
    idx_ref[:] = jnp.min(hit, axis=1).astype(jnp.int32)


def kernel(state, centroids):
    n, d = state.shape
    k, _ = centroids.shape
    grid = (n // N_BLOCK,)
    idx, dist = pl.pallas_call(
        _assign_body,
        grid=grid,
        in_specs=[
            pl.BlockSpec((N_BLOCK, d), lambda i: (i, 0)),
            pl.BlockSpec((k, d), lambda i: (0, 0)),
        ],
        out_specs=[
            pl.BlockSpec((N_BLOCK,), lambda i: (i,)),
            pl.BlockSpec((N_BLOCK,), lambda i: (i,)),
        ],
        out_shape=[
            jax.ShapeDtypeStruct((n,), jnp.int32),
            jax.ShapeDtypeStruct((n,), jnp.float32),
        ],
        compiler_params=pltpu.CompilerParams(
            dimension_semantics=("parallel",)),
    )(state, centroids)
    return (idx, dist)


# speedup vs baseline: 1.1869x; 1.0227x over previous
import jax
import jax.numpy as jnp
from jax.experimental import pallas as pl
from jax.experimental.pallas import tpu as pltpu

N_BLOCK = 2048


def _assign_body(state_ref, cent_ref, idx_ref, dist_ref):
    a = state_ref[:]
    c = cent_ref[:]
    x2 = jnp.sum(a * a, axis=1, keepdims=True)
    c2 = jnp.sum(c * c, axis=1)[None, :]
    mm = jax.lax.dot_general(
        a, c, (((1,), (1,)), ((), ())),
        preferred_element_type=jnp.float32)
    d = jnp.sqrt(jnp.clip((x2 + c2) - 2.0 * mm, 0.0, None))
    m = jnp.min(d, axis=1)
    dist_ref[:] = m
    iota = jax.lax.broadcasted_iota(jnp.int32, d.shape, 1).astype(jnp.float32)
    hit = jnp.where(d == m[:, None], iota, float(d.shape[1]))
    idx_ref[:] = jnp.min(hit, axis=1).astype(jnp.int32)


def kernel(state, centroids):
    n, d = state.shape
    k, _ = centroids.shape
    grid = (n // N_BLOCK,)
    idx, dist = pl.pallas_call(
        _assign_body,
        grid=grid,
        in_specs=[
            pl.BlockSpec((N_BLOCK, d), lambda i: (i, 0)),
            pl.BlockSpec((k, d), lambda i: (0, 0)),
        ],
        out_specs=[
            pl.BlockSpec((N_BLOCK,), lambda i: (i,)),
            pl.BlockSpec((N_BLOCK,), lambda i: (i,)),
        ],
        out_shape=[
            jax.ShapeDtypeStruct((n,), jnp.int32),
            jax.ShapeDtypeStruct((n,), jnp.float32),
        ],
        compiler_params=pltpu.CompilerParams(
            dimension_semantics=("parallel",)),
    )(state, centroids)
    return (idx, dist)


# speedup vs baseline: 1.4260x; 1.2015x over previous
import jax
import jax.numpy as jnp
from jax.experimental import pallas as pl
from jax.experimental.pallas import tpu as pltpu

N_BLOCK = 2048


def _assign_body(state_ref, cent_ref, idx_ref, dist_ref):
    a = state_ref[:]
    c = cent_ref[:]
    x2 = jnp.sum(a * a, axis=1, keepdims=True)
    c2 = jnp.sum(c * c, axis=1)[None, :]
    mm = jax.lax.dot_general(
        a, c, (((1,), (1,)), ((), ())),
        preferred_element_type=jnp.float32)
    d2 = jnp.clip((x2 + c2) - 2.0 * mm, 0.0, None)
    m2 = jnp.min(d2, axis=1)
    dist_ref[:] = jnp.sqrt(m2)
    iota = jax.lax.broadcasted_iota(jnp.int32, d2.shape, 1).astype(jnp.float32)
    hit = jnp.where(d2 == m2[:, None], iota, float(d2.shape[1]))
    idx_ref[:] = jnp.min(hit, axis=1).astype(jnp.int32)


def kernel(state, centroids):
    n, d = state.shape
    k, _ = centroids.shape
    grid = (n // N_BLOCK,)
    idx, dist = pl.pallas_call(
        _assign_body,
        grid=grid,
        in_specs=[
            pl.BlockSpec((N_BLOCK, d), lambda i: (i, 0)),
            pl.BlockSpec((k, d), lambda i: (0, 0)),
        ],
        out_specs=[
            pl.BlockSpec((N_BLOCK,), lambda i: (i,)),
            pl.BlockSpec((N_BLOCK,), lambda i: (i,)),
        ],
        out_shape=[
            jax.ShapeDtypeStruct((n,), jnp.int32),
            jax.ShapeDtypeStruct((n,), jnp.float32),
        ],
        compiler_params=pltpu.CompilerParams(
            dimension_semantics=("parallel",)),
    )(state, centroids)
    return (idx, dist)
